# initial kernel scaffold (unmeasured)
import jax
import jax.numpy as jnp
from jax import lax
from jax.experimental import pallas as pl
from jax.experimental.pallas import tpu as pltpu


def kernel(
    x,
):
    def body(*refs):
        pass

    out_shape = jax.ShapeDtypeStruct(..., jnp.float32)
    return pl.pallas_call(body, out_shape=out_shape)(...)


# baseline (device time: 194347 ns/iter reference)
import jax
import jax.numpy as jnp
from jax import lax
from jax.experimental import pallas as pl
from jax.experimental.pallas import tpu as pltpu


def kernel(x):
    x = x.reshape(x.shape[-2], x.shape[-1])
    m, n = x.shape

    def body(x_ref, out_ref, comm_ref, send_sems, recv_sems):
        my_x = lax.axis_index("x")
        my_y = lax.axis_index("y")
        x_nbr = (1 - my_x, my_y)
        y_nbr = (my_x, 1 - my_y)

        barrier_sem = pltpu.get_barrier_semaphore()
        for nbr in (x_nbr, y_nbr):
            pl.semaphore_signal(
                barrier_sem, inc=1,
                device_id=nbr, device_id_type=pl.DeviceIdType.MESH,
            )
        pl.semaphore_wait(barrier_sem, 2)

        rdma1 = pltpu.make_async_remote_copy(
            src_ref=x_ref,
            dst_ref=comm_ref.at[0],
            send_sem=send_sems.at[0],
            recv_sem=recv_sems.at[0],
            device_id=y_nbr,
            device_id_type=pl.DeviceIdType.MESH,
        )
        rdma1.start()
        rdma1.wait()
        out_ref[...] = x_ref[...] + comm_ref[0]

        rdma2 = pltpu.make_async_remote_copy(
            src_ref=out_ref,
            dst_ref=comm_ref.at[1],
            send_sem=send_sems.at[1],
            recv_sem=recv_sems.at[1],
            device_id=x_nbr,
            device_id_type=pl.DeviceIdType.MESH,
        )
        rdma2.start()
        rdma2.wait()
        out_ref[...] = out_ref[...] + comm_ref[1]

    return pl.pallas_call(
        body,
        out_shape=jax.ShapeDtypeStruct((m, n), jnp.float32),
        in_specs=[pl.BlockSpec(memory_space=pltpu.VMEM)],
        out_specs=pl.BlockSpec(memory_space=pltpu.VMEM),
        scratch_shapes=[
            pltpu.VMEM((2, m, n), jnp.float32),
            pltpu.SemaphoreType.DMA((2,)),
            pltpu.SemaphoreType.DMA((2,)),
        ],
        compiler_params=pltpu.CompilerParams(collective_id=0),
    )(x)


# device time: 83838 ns/iter; 2.3181x vs baseline; 2.3181x over previous
import jax
import jax.numpy as jnp
from jax import lax
from jax.experimental import pallas as pl
from jax.experimental.pallas import tpu as pltpu


def kernel(x):
    x = x.reshape(x.shape[-2], x.shape[-1])
    m, n = x.shape
    half = m // 2
    q = m // 4
    e = m // 8

    def body(x_ref, out_ref, comm1, comm2, send_sems, recv_sems):
        my_x = lax.axis_index("x")
        my_y = lax.axis_index("y")
        x_nbr = (1 - my_x, my_y)
        y_nbr = (my_x, 1 - my_y)

        barrier_sem = pltpu.get_barrier_semaphore()
        for nbr in (x_nbr, y_nbr):
            pl.semaphore_signal(
                barrier_sem, inc=1,
                device_id=nbr, device_id_type=pl.DeviceIdType.MESH,
            )
        pl.semaphore_wait(barrier_sem, 2)

        a_base = my_y * q
        b_base = half + my_x * q

        def exch(src, dst, sem, nbr):
            return pltpu.make_async_remote_copy(
                src_ref=src, dst_ref=dst,
                send_sem=send_sems.at[sem], recv_sem=recv_sems.at[sem],
                device_id=nbr, device_id_type=pl.DeviceIdType.MESH,
            )

        a1 = exch(x_ref.at[pl.ds((1 - my_y) * q, q), :], comm1.at[0], 0, y_nbr)
        b1 = exch(x_ref.at[pl.ds(half + (1 - my_x) * q, q), :], comm1.at[1], 1, x_nbr)
        a1.start()
        b1.start()

        a1.wait()
        out_ref[pl.ds(a_base, q), :] = x_ref[pl.ds(a_base, q), :] + comm1[0]
        a2 = exch(out_ref.at[pl.ds(a_base + (1 - my_x) * e, e), :],
                  comm2.at[0], 2, x_nbr)
        a2.start()

        b1.wait()
        out_ref[pl.ds(b_base, q), :] = x_ref[pl.ds(b_base, q), :] + comm1[1]
        b2 = exch(out_ref.at[pl.ds(b_base + (1 - my_y) * e, e), :],
                  comm2.at[1], 3, y_nbr)
        b2.start()

        a2.wait()
        a_mine = a_base + my_x * e
        out_ref[pl.ds(a_mine, e), :] = out_ref[pl.ds(a_mine, e), :] + comm2[0]
        a3 = exch(out_ref.at[pl.ds(a_mine, e), :],
                  out_ref.at[pl.ds(a_mine, e), :], 4, x_nbr)
        a3.start()

        b2.wait()
        b_mine = b_base + my_y * e
        out_ref[pl.ds(b_mine, e), :] = out_ref[pl.ds(b_mine, e), :] + comm2[1]
        b3 = exch(out_ref.at[pl.ds(b_mine, e), :],
                  out_ref.at[pl.ds(b_mine, e), :], 5, y_nbr)
        b3.start()

        a3.wait()
        a4 = exch(out_ref.at[pl.ds(a_base, q), :],
                  out_ref.at[pl.ds(a_base, q), :], 6, y_nbr)
        a4.start()

        b3.wait()
        b4 = exch(out_ref.at[pl.ds(b_base, q), :],
                  out_ref.at[pl.ds(b_base, q), :], 7, x_nbr)
        b4.start()

        a4.wait()
        b4.wait()

    return pl.pallas_call(
        body,
        out_shape=jax.ShapeDtypeStruct((m, n), jnp.float32),
        in_specs=[pl.BlockSpec(memory_space=pltpu.VMEM)],
        out_specs=pl.BlockSpec(memory_space=pltpu.VMEM),
        scratch_shapes=[
            pltpu.VMEM((2, q, n), jnp.float32),
            pltpu.VMEM((2, e, n), jnp.float32),
            pltpu.SemaphoreType.DMA((8,)),
            pltpu.SemaphoreType.DMA((8,)),
        ],
        compiler_params=pltpu.CompilerParams(collective_id=0),
    )(x)


# device time: 82551 ns/iter; 2.3543x vs baseline; 1.0156x over previous
import jax
import jax.numpy as jnp
from jax import lax
from jax.experimental import pallas as pl
from jax.experimental.pallas import tpu as pltpu


def kernel(x):
    x = x.reshape(x.shape[-2], x.shape[-1])
    m, n = x.shape
    half = m // 2
    q = m // 4
    e = m // 8

    def body(x_ref, out_ref, comm1, comm2, send_sems, recv_sems):
        my_x = lax.axis_index("x")
        my_y = lax.axis_index("y")
        x_nbr = (1 - my_x, my_y)
        y_nbr = (my_x, 1 - my_y)

        barrier_sem = pltpu.get_barrier_semaphore()
        for nbr in (x_nbr, y_nbr):
            pl.semaphore_signal(
                barrier_sem, inc=1,
                device_id=nbr, device_id_type=pl.DeviceIdType.MESH,
            )
        pl.semaphore_wait(barrier_sem, 2)

        a_base = my_y * q
        b_base = half + my_x * q
        ja0 = 1 - my_x
        jb0 = 1 - my_y
        a_mine = a_base + my_x * e
        b_mine = b_base + my_y * e

        def exch(src, dst, sem, nbr):
            return pltpu.make_async_remote_copy(
                src_ref=src, dst_ref=dst,
                send_sem=send_sems.at[sem], recv_sem=recv_sems.at[sem],
                device_id=nbr, device_id_type=pl.DeviceIdType.MESH,
            )

        a1p0 = exch(x_ref.at[pl.ds((1 - my_y) * q + ja0 * e, e), :],
                    comm1.at[0], 0, y_nbr)
        a1p1 = exch(x_ref.at[pl.ds((1 - my_y) * q + (1 - ja0) * e, e), :],
                    comm1.at[1], 1, y_nbr)
        b1p0 = exch(x_ref.at[pl.ds(half + (1 - my_x) * q + jb0 * e, e), :],
                    comm1.at[2], 2, x_nbr)
        b1p1 = exch(x_ref.at[pl.ds(half + (1 - my_x) * q + (1 - jb0) * e, e), :],
                    comm1.at[3], 3, x_nbr)
        a1p0.start()
        a1p1.start()
        b1p0.start()
        b1p1.start()

        a1p0.wait()
        out_ref[pl.ds(a_base + ja0 * e, e), :] = (
            x_ref[pl.ds(a_base + ja0 * e, e), :] + comm1[0]
        )
        a2 = exch(out_ref.at[pl.ds(a_base + ja0 * e, e), :],
                  comm2.at[0], 4, x_nbr)
        a2.start()

        b1p0.wait()
        out_ref[pl.ds(b_base + jb0 * e, e), :] = (
            x_ref[pl.ds(b_base + jb0 * e, e), :] + comm1[2]
        )
        b2 = exch(out_ref.at[pl.ds(b_base + jb0 * e, e), :],
                  comm2.at[1], 5, y_nbr)
        b2.start()

        a1p1.wait()
        out_ref[pl.ds(a_base + (1 - ja0) * e, e), :] = (
            x_ref[pl.ds(a_base + (1 - ja0) * e, e), :] + comm1[1]
        )
        b1p1.wait()
        out_ref[pl.ds(b_base + (1 - jb0) * e, e), :] = (
            x_ref[pl.ds(b_base + (1 - jb0) * e, e), :] + comm1[3]
        )

        a2.wait()
        out_ref[pl.ds(a_mine, e), :] = out_ref[pl.ds(a_mine, e), :] + comm2[0]
        a3 = exch(out_ref.at[pl.ds(a_mine, e), :],
                  out_ref.at[pl.ds(a_mine, e), :], 6, x_nbr)
        a3.start()
        a4a = exch(out_ref.at[pl.ds(a_mine, e), :],
                   out_ref.at[pl.ds(a_mine, e), :], 8, y_nbr)
        a4a.start()

        b2.wait()
        out_ref[pl.ds(b_mine, e), :] = out_ref[pl.ds(b_mine, e), :] + comm2[1]
        b3 = exch(out_ref.at[pl.ds(b_mine, e), :],
                  out_ref.at[pl.ds(b_mine, e), :], 7, y_nbr)
        b3.start()
        b4a = exch(out_ref.at[pl.ds(b_mine, e), :],
                   out_ref.at[pl.ds(b_mine, e), :], 10, x_nbr)
        b4a.start()

        a3.wait()
        a4b = exch(out_ref.at[pl.ds(a_base + (1 - my_x) * e, e), :],
                   out_ref.at[pl.ds(a_base + (1 - my_x) * e, e), :], 9, y_nbr)
        a4b.start()

        b3.wait()
        b4b = exch(out_ref.at[pl.ds(b_base + (1 - my_y) * e, e), :],
                   out_ref.at[pl.ds(b_base + (1 - my_y) * e, e), :], 11, x_nbr)
        b4b.start()

        a4a.wait()
        a4b.wait()
        b4a.wait()
        b4b.wait()

    return pl.pallas_call(
        body,
        out_shape=jax.ShapeDtypeStruct((m, n), jnp.float32),
        in_specs=[pl.BlockSpec(memory_space=pltpu.VMEM)],
        out_specs=pl.BlockSpec(memory_space=pltpu.VMEM),
        scratch_shapes=[
            pltpu.VMEM((4, e, n), jnp.float32),
            pltpu.VMEM((2, e, n), jnp.float32),
            pltpu.SemaphoreType.DMA((12,)),
            pltpu.SemaphoreType.DMA((12,)),
        ],
        compiler_params=pltpu.CompilerParams(collective_id=0),
    )(x)
